# Initial kernel scaffold; baseline (speedup 1.0000x reference)
import jax
import jax.numpy as jnp
from jax.experimental import pallas as pl


def kernel(input_seq, input_seq_timestamp, tgt_idx, edge_index, emb_table, gcn_w1, gcn_w2, global_snaps, cas_snaps, pos_emb, attn_w, ffn_w1, ffn_b1, ffn_w2, ffn_b2, ln_p, fus2_l1w, fus2_l1b, fus2_l2w, fus2_l2b, fus_l1w, fus_l1b, fus_l2w, fus_l2b, lin2_w, lin2_b):
    raise NotImplementedError("write your pallas kernel here")


# speedup vs baseline: 2.3164x; 2.3164x over previous
import functools

import jax
import jax.numpy as jnp
import numpy as np
from jax.experimental import pallas as pl

N_NODE = 10000
D_MODEL = 128
POS_DIM = 8
N_HEADS = 8
B = 16
L = 99
D = D_MODEL + POS_DIM
D_FF = 4 * D

COL_BLK = 2048


def _final_proj_body(att_ref, w_ref, b_ref, seq_ref, out_ref):
    c = pl.program_id(1)
    a = att_ref[0]
    w = w_ref[...]
    acc = jnp.dot(a, w, preferred_element_type=jnp.float32)
    acc = acc + b_ref[...]
    cols = c * COL_BLK + jax.lax.broadcasted_iota(jnp.int32, (1, COL_BLK), 1)
    s = seq_ref[0]
    hit = (s.reshape(L, 1) == cols).astype(jnp.float32)
    ri = jax.lax.broadcasted_iota(jnp.int32, (L, L), 0)
    ci = jax.lax.broadcasted_iota(jnp.int32, (L, L), 1)
    tril = (ri >= ci).astype(jnp.float32)
    cum = jnp.dot(tril, hit, preferred_element_type=jnp.float32)
    li = jax.lax.broadcasted_iota(jnp.int32, (L, COL_BLK), 0)
    masked = (cum > 0.0) | ((cols == 0) & (li < L - 1))
    out_ref[0] = acc + jnp.where(masked, -1000.0, 0.0)


def _final_projection(att, lin2_w, lin2_b, seq):
    n_cb = pl.cdiv(N_NODE, COL_BLK)
    out = pl.pallas_call(
        _final_proj_body,
        grid=(B, n_cb),
        in_specs=[
            pl.BlockSpec((1, L, D), lambda b, c: (b, 0, 0)),
            pl.BlockSpec((D, COL_BLK), lambda b, c: (0, c)),
            pl.BlockSpec((1, COL_BLK), lambda b, c: (0, c)),
            pl.BlockSpec((1, 1, L), lambda b, c: (b, 0, 0)),
        ],
        out_specs=pl.BlockSpec((1, L, COL_BLK), lambda b, c: (b, 0, c)),
        out_shape=jax.ShapeDtypeStruct((B, L, N_NODE), jnp.float32),
    )(att, lin2_w, lin2_b.reshape(1, N_NODE), seq.reshape(B, 1, L))
    return out.reshape(B * L, N_NODE)


def _layer_norm(x, s, b):
    mu = jnp.mean(x, axis=-1, keepdims=True)
    var = jnp.var(x, axis=-1, keepdims=True)
    return (x - mu) * jax.lax.rsqrt(var + 1e-6) * s + b


def _block(x, pad_mask, w, w1, b1, w2, b2, lnp):
    Bx, Lx, Dx = x.shape
    dk = Dx // N_HEADS

    def proj(Wm):
        return (x @ Wm).reshape(Bx, Lx, N_HEADS, dk).transpose(0, 2, 1, 3)

    q, k, v = proj(w[0]), proj(w[1]), proj(w[2])
    sc = jnp.einsum('bhqd,bhkd->bhqk', q, k) / np.sqrt(dk)
    sc = jnp.where(pad_mask[:, None, None, :], -1e9, sc)
    a = jax.nn.softmax(sc, axis=-1)
    o = jnp.einsum('bhqk,bhkd->bhqd', a, v).transpose(0, 2, 1, 3).reshape(Bx, Lx, Dx) @ w[3]
    x1 = _layer_norm(x + o, lnp[0, 0], lnp[0, 1])
    ff = jax.nn.relu(x1 @ w1 + b1) @ w2 + b2
    return _layer_norm(x1 + ff, lnp[1, 0], lnp[1, 1])


def _fusion(a, b, l1w, l1b, l2w, l2b):
    emb = jnp.stack([a, b], axis=0)
    score = jax.nn.softmax(jnp.tanh(emb @ l1w + l1b) @ l2w + l2b, axis=0)
    return jnp.sum(score * emb, axis=0)


def kernel(input_seq, input_seq_timestamp, tgt_idx, edge_index, emb_table,
           gcn_w1, gcn_w2, global_snaps, cas_snaps, pos_emb, attn_w,
           ffn_w1, ffn_b1, ffn_w2, ffn_b2, ln_p, fus2_l1w, fus2_l1b,
           fus2_l2w, fus2_l2b, fus_l1w, fus_l1b, fus_l2w, fus_l2b,
           lin2_w, lin2_b):
    seq = input_seq[:, :-1]
    ts = input_seq_timestamp[:, :-1]
    pad_mask = seq == 0
    src, dst = edge_index[0], edge_index[1]

    deg = jnp.zeros((N_NODE,), jnp.float32).at[dst].add(1.0) + 1.0
    norm = jax.lax.rsqrt(deg)
    xn1 = emb_table * norm[:, None]
    agg1 = jnp.zeros_like(xn1).at[dst].add(xn1[src])
    xn2 = (norm * norm)[:, None] * (agg1 + xn1)
    agg2 = jnp.zeros_like(xn2).at[dst].add(xn2[src])
    z = norm[:, None] * (agg2 + xn2)

    w12 = gcn_w1 @ gcn_w2
    zg = z[seq]
    hseq = zg @ w12

    nz = seq != 0
    gflat = global_snaps.reshape(3 * N_NODE, D_MODEL)
    gidx = (jnp.clip(ts, 1, 3) - 1) * N_NODE + seq
    gsel = gflat[gidx]
    is0 = (ts == 0) & nz
    dyemb = jnp.where(is0[..., None], hseq,
                      jnp.where(nz[..., None], gsel, 0.0))
    c0 = cas_snaps[0][tgt_idx]
    c1 = cas_snaps[1][tgt_idx]
    casr = jnp.where((ts == 1)[..., None], c0[:, None, :], c1[:, None, :])
    cas_emb = jnp.where(is0[..., None], hseq,
                        jnp.where(nz[..., None], casr, 0.0))

    dyf = _fusion(dyemb, cas_emb, fus2_l1w, fus2_l1b, fus2_l2w, fus2_l2b)
    order = jnp.broadcast_to(pos_emb[:L][None], (B, L, POS_DIM))
    diff_embed = jnp.concatenate([dyf, order], axis=-1)
    fri_embed = jnp.concatenate([hseq, order], axis=-1)

    d_att = _block(diff_embed, pad_mask, attn_w[0], ffn_w1[0], ffn_b1[0],
                   ffn_w2[0], ffn_b2[0], ln_p[0])
    f_att = _block(fri_embed, pad_mask, attn_w[1], ffn_w1[1], ffn_b1[1],
                   ffn_w2[1], ffn_b2[1], ln_p[1])
    att = _fusion(d_att, f_att, fus_l1w, fus_l1b, fus_l2w, fus_l2b)

    return _final_projection(att, lin2_w, lin2_b, seq)


# speedup vs baseline: 2.3743x; 1.0250x over previous
import functools

import jax
import jax.numpy as jnp
import numpy as np
from jax.experimental import pallas as pl

N_NODE = 10000
D_MODEL = 128
POS_DIM = 8
N_HEADS = 8
B = 16
L = 99
D = D_MODEL + POS_DIM
DK = D // N_HEADS
D_FF = 4 * D


def _ln(x, s, b):
    mu = jnp.mean(x, axis=1, keepdims=True)
    var = jnp.mean((x - mu) * (x - mu), axis=1, keepdims=True)
    return (x - mu) * jax.lax.rsqrt(var + 1e-6) * s + b


def _mega_body(seqT, seqR, tsT, zg, gsel, c0r, c1r, pos, gw1, gw2, aw,
               fw1, fb1, fw2, fb2, lnp, f2l1w, f2l1b, f2l2w, f2l2b,
               fl1w, fl1b, fl2w, fl2b, l2w, l2b, out):
    f32 = jnp.float32
    w12 = jnp.dot(gw1[...], gw2[...], preferred_element_type=f32)
    hseq = jnp.dot(zg[0], w12, preferred_element_type=f32)

    st = seqT[0]
    tt = tsT[0]
    nz = st != 0
    is0 = nz & (tt == 0)
    dy = jnp.where(is0, hseq, jnp.where(nz, gsel[0], 0.0))
    casr = jnp.where(tt == 1, c0r[0], c1r[0])
    cas = jnp.where(is0, hseq, jnp.where(nz, casr, 0.0))

    def fuse(a, b, l1w, l1b, l2w_, l2b_):
        sa = jnp.dot(jnp.tanh(jnp.dot(a, l1w, preferred_element_type=f32) + l1b),
                     l2w_, preferred_element_type=f32) + l2b_
        sb = jnp.dot(jnp.tanh(jnp.dot(b, l1w, preferred_element_type=f32) + l1b),
                     l2w_, preferred_element_type=f32) + l2b_
        m = jnp.maximum(sa, sb)
        ea = jnp.exp(sa - m)
        eb = jnp.exp(sb - m)
        den = ea + eb
        return (ea / den) * a + (eb / den) * b

    dyf = fuse(dy, cas, f2l1w[...], f2l1b[...], f2l2w[...], f2l2b[...])
    o99 = pos[...]
    diff = jnp.concatenate([dyf, o99], axis=1)
    fri = jnp.concatenate([hseq, o99], axis=1)

    padR = seqR[0] == 0
    lane = jax.lax.broadcasted_iota(jnp.int32, (1, D), 1)
    scale = 1.0 / np.sqrt(DK)

    def blk(x, bi):
        wq, wk, wv, wo = aw[bi, 0], aw[bi, 1], aw[bi, 2], aw[bi, 3]
        q = jnp.dot(x, wq, preferred_element_type=f32)
        k = jnp.dot(x, wk, preferred_element_type=f32)
        v = jnp.dot(x, wv, preferred_element_type=f32)
        o = jnp.zeros_like(x)
        for h in range(N_HEADS):
            mh = ((lane >= h * DK) & (lane < (h + 1) * DK)).astype(f32)
            s = jax.lax.dot_general(q * mh, k, (((1,), (1,)), ((), ())),
                                    preferred_element_type=f32) * scale
            s = jnp.where(padR, -1e9, s)
            m = jnp.max(s, axis=1, keepdims=True)
            e = jnp.exp(s - m)
            a = e / jnp.sum(e, axis=1, keepdims=True)
            o = o + jnp.dot(a, v * mh, preferred_element_type=f32)
        o = jnp.dot(o, wo, preferred_element_type=f32)
        x1 = _ln(x + o, lnp[bi, 0, 0], lnp[bi, 0, 1])
        ff = jnp.maximum(jnp.dot(x1, fw1[bi], preferred_element_type=f32) + fb1[bi], 0.0)
        ff = jnp.dot(ff, fw2[bi], preferred_element_type=f32) + fb2[bi]
        return _ln(x1 + ff, lnp[bi, 1, 0], lnp[bi, 1, 1])

    datt = blk(diff, 0)
    fatt = blk(fri, 1)
    att = fuse(datt, fatt, fl1w[...], fl1b[...], fl2w[...], fl2b[...])

    acc = jnp.dot(att, l2w[...], preferred_element_type=f32) + l2b[...]
    cols = jax.lax.broadcasted_iota(jnp.int32, (1, N_NODE), 1)
    hit = (st == cols).astype(f32)
    ri = jax.lax.broadcasted_iota(jnp.int32, (L, L), 0)
    ci = jax.lax.broadcasted_iota(jnp.int32, (L, L), 1)
    tril = (ri >= ci).astype(f32)
    cum = jnp.dot(tril, hit, preferred_element_type=f32)
    li = jax.lax.broadcasted_iota(jnp.int32, (L, 1), 0)
    masked = (cum > 0.0) | ((cols == 0) & (li < L - 1))
    out[0] = acc + jnp.where(masked, -1000.0, 0.0)


def _const(shape):
    nd = len(shape)
    return pl.BlockSpec(shape, lambda b, _n=nd: (0,) * _n)


def _mega(seqT, seqR, tsT, zg, gsel, c0g, c1g, pos99, gcn_w1, gcn_w2,
          attn_w, ffn_w1, ffn_b1, ffn_w2, ffn_b2, ln_p, fus2_l1w,
          fus2_l1b, fus2_l2w, fus2_l2b, fus_l1w, fus_l1b, fus_l2w,
          fus_l2b, lin2_w, lin2_b):
    out = pl.pallas_call(
        _mega_body,
        grid=(B,),
        in_specs=[
            pl.BlockSpec((1, L, 1), lambda b: (b, 0, 0)),
            pl.BlockSpec((1, 1, L), lambda b: (b, 0, 0)),
            pl.BlockSpec((1, L, 1), lambda b: (b, 0, 0)),
            pl.BlockSpec((1, L, D_MODEL), lambda b: (b, 0, 0)),
            pl.BlockSpec((1, L, D_MODEL), lambda b: (b, 0, 0)),
            pl.BlockSpec((1, 1, D_MODEL), lambda b: (b, 0, 0)),
            pl.BlockSpec((1, 1, D_MODEL), lambda b: (b, 0, 0)),
            _const((L, POS_DIM)),
            _const((D_MODEL, 2 * D_MODEL)),
            _const((2 * D_MODEL, D_MODEL)),
            _const((2, 4, D, D)),
            _const((2, D, D_FF)),
            _const((2, D_FF)),
            _const((2, D_FF, D)),
            _const((2, D)),
            _const((2, 2, 2, D)),
            _const((D_MODEL, D_MODEL)),
            _const((D_MODEL,)),
            _const((D_MODEL, 1)),
            _const((1, 1)),
            _const((D, D)),
            _const((D,)),
            _const((D, 1)),
            _const((1, 1)),
            _const((D, N_NODE)),
            _const((1, N_NODE)),
        ],
        out_specs=pl.BlockSpec((1, L, N_NODE), lambda b: (b, 0, 0)),
        out_shape=jax.ShapeDtypeStruct((B, L, N_NODE), jnp.float32),
    )(seqT, seqR, tsT, zg, gsel, c0g, c1g, pos99, gcn_w1, gcn_w2,
      attn_w, ffn_w1, ffn_b1, ffn_w2, ffn_b2, ln_p, fus2_l1w, fus2_l1b,
      fus2_l2w, fus2_l2b, fus_l1w, fus_l1b, fus_l2w, fus_l2b,
      lin2_w, lin2_b)
    return out.reshape(B * L, N_NODE)


def kernel(input_seq, input_seq_timestamp, tgt_idx, edge_index, emb_table,
           gcn_w1, gcn_w2, global_snaps, cas_snaps, pos_emb, attn_w,
           ffn_w1, ffn_b1, ffn_w2, ffn_b2, ln_p, fus2_l1w, fus2_l1b,
           fus2_l2w, fus2_l2b, fus_l1w, fus_l1b, fus_l2w, fus_l2b,
           lin2_w, lin2_b):
    seq = input_seq[:, :-1]
    ts = input_seq_timestamp[:, :-1]
    src, dst = edge_index[0], edge_index[1]

    deg = jnp.zeros((N_NODE,), jnp.float32).at[dst].add(1.0) + 1.0
    norm = jax.lax.rsqrt(deg)
    xn1 = emb_table * norm[:, None]
    agg1 = jnp.zeros_like(xn1).at[dst].add(xn1[src])
    xn2 = (norm * norm)[:, None] * (agg1 + xn1)
    agg2 = jnp.zeros_like(xn2).at[dst].add(xn2[src])
    z = norm[:, None] * (agg2 + xn2)

    zg = z[seq]
    gflat = global_snaps.reshape(3 * N_NODE, D_MODEL)
    gidx = (jnp.clip(ts, 1, 3) - 1) * N_NODE + seq
    gsel = gflat[gidx]
    c0g = cas_snaps[0][tgt_idx][:, None, :]
    c1g = cas_snaps[1][tgt_idx][:, None, :]

    return _mega(seq[:, :, None], seq[:, None, :], ts[:, :, None],
                 zg, gsel, c0g, c1g, pos_emb[:L], gcn_w1, gcn_w2,
                 attn_w, ffn_w1, ffn_b1, ffn_w2, ffn_b2, ln_p,
                 fus2_l1w, fus2_l1b, fus2_l2w, fus2_l2b.reshape(1, 1),
                 fus_l1w, fus_l1b, fus_l2w, fus_l2b.reshape(1, 1),
                 lin2_w, lin2_b.reshape(1, N_NODE))


# speedup vs baseline: 9.3444x; 3.9357x over previous
import functools

import jax
import jax.numpy as jnp
import numpy as np
from jax import lax
from jax.experimental import pallas as pl
from jax.experimental.pallas import tpu as pltpu
from jax.experimental.pallas import tpu_sc as plsc

N_NODE = 10000
D_MODEL = 128
POS_DIM = 8
N_HEADS = 8
B = 16
L = 99
D = D_MODEL + POS_DIM
DK = D // N_HEADS
D_FF = 4 * D


N_EDGES = 160000
NW = 32
EPT = N_EDGES // NW
CH = 128
NCH = EPT // CH
REM = EPT - NCH * CH
NPAD = 10240
RPT = NPAD // 16


def _sc_agg_body(x_hbm, src_hbm, dst_hbm, zero_hbm, out_hbm,
                 idxs, idxd, rows, idxs_r, idxd_r, rows_r, aggs, sem):
    c = lax.axis_index("c")
    s = lax.axis_index("s")
    w = s * 2 + c
    base = w * EPT
    pltpu.sync_copy(zero_hbm.at[pl.ds(s * RPT, RPT)],
                    aggs.at[pl.ds(s * RPT, RPT)])
    plsc.subcore_barrier()

    def body(j, carry):
        off = base + j * CH
        pltpu.sync_copy(src_hbm.at[pl.ds(off, CH)], idxs)
        pltpu.sync_copy(dst_hbm.at[pl.ds(off, CH)], idxd)
        pltpu.async_copy(x_hbm.at[idxs], rows, sem).wait()
        pltpu.sync_copy(rows, aggs.at[idxd], add=True)
        return carry

    lax.fori_loop(0, NCH, body, 0)
    off = base + NCH * CH
    pltpu.sync_copy(src_hbm.at[pl.ds(off, REM)], idxs_r)
    pltpu.sync_copy(dst_hbm.at[pl.ds(off, REM)], idxd_r)
    pltpu.async_copy(x_hbm.at[idxs_r], rows_r, sem).wait()
    pltpu.sync_copy(rows_r, aggs.at[idxd_r], add=True)
    plsc.subcore_barrier()
    pltpu.sync_copy(aggs.at[pl.ds(s * RPT, RPT)],
                    out_hbm.at[c, pl.ds(s * RPT, RPT)])


def _sc_agg(xn, src, dst, zero_big):
    f = pl.kernel(
        _sc_agg_body,
        out_type=jax.ShapeDtypeStruct((2, NPAD, D_MODEL), jnp.float32),
        mesh=plsc.VectorSubcoreMesh(core_axis_name="c", subcore_axis_name="s"),
        scratch_types=[
            pltpu.VMEM((CH,), jnp.int32),
            pltpu.VMEM((CH,), jnp.int32),
            pltpu.VMEM((CH, D_MODEL), jnp.float32),
            pltpu.VMEM((REM,), jnp.int32),
            pltpu.VMEM((REM,), jnp.int32),
            pltpu.VMEM((REM, D_MODEL), jnp.float32),
            pltpu.VMEM_SHARED((NPAD, D_MODEL), jnp.float32),
            pltpu.SemaphoreType.DMA,
        ],
    )
    p = f(xn, src, dst, zero_big)
    return p[0, :N_NODE] + p[1, :N_NODE]


def _sc_deg_body(dst_hbm, zero_hbm, ones_hbm, out_hbm,
                 idxd, ones_v, idxd_r, ones_r, hist):
    c = lax.axis_index("c")
    s = lax.axis_index("s")
    w = s * 2 + c
    base = w * EPT
    pltpu.sync_copy(zero_hbm.at[pl.ds(s * RPT, RPT)],
                    hist.at[pl.ds(s * RPT, RPT)])
    pltpu.sync_copy(ones_hbm, ones_v)
    pltpu.sync_copy(ones_hbm.at[pl.ds(0, REM)], ones_r)
    plsc.subcore_barrier()

    def body(j, carry):
        off = base + j * CH
        pltpu.sync_copy(dst_hbm.at[pl.ds(off, CH)], idxd)
        pltpu.sync_copy(ones_v, hist.at[idxd], add=True)
        return carry

    lax.fori_loop(0, NCH, body, 0)
    off = base + NCH * CH
    pltpu.sync_copy(dst_hbm.at[pl.ds(off, REM)], idxd_r)
    pltpu.sync_copy(ones_r, hist.at[idxd_r], add=True)
    plsc.subcore_barrier()
    pltpu.sync_copy(hist.at[pl.ds(s * RPT, RPT)],
                    out_hbm.at[c, pl.ds(s * RPT, RPT)])


def _sc_deg(dst, zero_col, ones_col):
    f = pl.kernel(
        _sc_deg_body,
        out_type=jax.ShapeDtypeStruct((2, NPAD), jnp.float32),
        mesh=plsc.VectorSubcoreMesh(core_axis_name="c", subcore_axis_name="s"),
        scratch_types=[
            pltpu.VMEM((CH,), jnp.int32),
            pltpu.VMEM((CH,), jnp.float32),
            pltpu.VMEM((REM,), jnp.int32),
            pltpu.VMEM((REM,), jnp.float32),
            pltpu.VMEM_SHARED((NPAD,), jnp.float32),
        ],
    )
    p = f(dst, zero_col, ones_col)
    return p[0, :N_NODE] + p[1, :N_NODE]


def _ln(x, s, b):
    mu = jnp.mean(x, axis=1, keepdims=True)
    var = jnp.mean((x - mu) * (x - mu), axis=1, keepdims=True)
    return (x - mu) * jax.lax.rsqrt(var + 1e-6) * s + b


def _mega_body(seqT, seqR, tsT, zg, gsel, c0r, c1r, pos, gw1, gw2, aw,
               fw1, fb1, fw2, fb2, lnp, f2l1w, f2l1b, f2l2w, f2l2b,
               fl1w, fl1b, fl2w, fl2b, l2w, l2b, out):
    f32 = jnp.float32
    w12 = jnp.dot(gw1[...], gw2[...], preferred_element_type=f32)
    hseq = jnp.dot(zg[0], w12, preferred_element_type=f32)

    st = seqT[0]
    tt = tsT[0]
    nz = st != 0
    is0 = nz & (tt == 0)
    dy = jnp.where(is0, hseq, jnp.where(nz, gsel[0], 0.0))
    casr = jnp.where(tt == 1, c0r[0], c1r[0])
    cas = jnp.where(is0, hseq, jnp.where(nz, casr, 0.0))

    def fuse(a, b, l1w, l1b, l2w_, l2b_):
        sa = jnp.dot(jnp.tanh(jnp.dot(a, l1w, preferred_element_type=f32) + l1b),
                     l2w_, preferred_element_type=f32) + l2b_
        sb = jnp.dot(jnp.tanh(jnp.dot(b, l1w, preferred_element_type=f32) + l1b),
                     l2w_, preferred_element_type=f32) + l2b_
        m = jnp.maximum(sa, sb)
        ea = jnp.exp(sa - m)
        eb = jnp.exp(sb - m)
        den = ea + eb
        return (ea / den) * a + (eb / den) * b

    dyf = fuse(dy, cas, f2l1w[...], f2l1b[...], f2l2w[...], f2l2b[...])
    o99 = pos[...]
    diff = jnp.concatenate([dyf, o99], axis=1)
    fri = jnp.concatenate([hseq, o99], axis=1)

    padR = seqR[0] == 0
    lane = jax.lax.broadcasted_iota(jnp.int32, (1, D), 1)
    scale = 1.0 / np.sqrt(DK)

    def blk(x, bi):
        wq, wk, wv, wo = aw[bi, 0], aw[bi, 1], aw[bi, 2], aw[bi, 3]
        q = jnp.dot(x, wq, preferred_element_type=f32)
        k = jnp.dot(x, wk, preferred_element_type=f32)
        v = jnp.dot(x, wv, preferred_element_type=f32)
        o = jnp.zeros_like(x)
        for h in range(N_HEADS):
            mh = ((lane >= h * DK) & (lane < (h + 1) * DK)).astype(f32)
            s = jax.lax.dot_general(q * mh, k, (((1,), (1,)), ((), ())),
                                    preferred_element_type=f32) * scale
            s = jnp.where(padR, -1e9, s)
            m = jnp.max(s, axis=1, keepdims=True)
            e = jnp.exp(s - m)
            a = e / jnp.sum(e, axis=1, keepdims=True)
            o = o + jnp.dot(a, v * mh, preferred_element_type=f32)
        o = jnp.dot(o, wo, preferred_element_type=f32)
        x1 = _ln(x + o, lnp[bi, 0, 0], lnp[bi, 0, 1])
        ff = jnp.maximum(jnp.dot(x1, fw1[bi], preferred_element_type=f32) + fb1[bi], 0.0)
        ff = jnp.dot(ff, fw2[bi], preferred_element_type=f32) + fb2[bi]
        return _ln(x1 + ff, lnp[bi, 1, 0], lnp[bi, 1, 1])

    datt = blk(diff, 0)
    fatt = blk(fri, 1)
    att = fuse(datt, fatt, fl1w[...], fl1b[...], fl2w[...], fl2b[...])

    acc = jnp.dot(att, l2w[...], preferred_element_type=f32) + l2b[...]
    cols = jax.lax.broadcasted_iota(jnp.int32, (1, N_NODE), 1)
    hit = (st == cols).astype(f32)
    ri = jax.lax.broadcasted_iota(jnp.int32, (L, L), 0)
    ci = jax.lax.broadcasted_iota(jnp.int32, (L, L), 1)
    tril = (ri >= ci).astype(f32)
    cum = jnp.dot(tril, hit, preferred_element_type=f32)
    li = jax.lax.broadcasted_iota(jnp.int32, (L, 1), 0)
    masked = (cum > 0.0) | ((cols == 0) & (li < L - 1))
    out[0] = acc + jnp.where(masked, -1000.0, 0.0)


def _const(shape):
    nd = len(shape)
    return pl.BlockSpec(shape, lambda b, _n=nd: (0,) * _n)


def _mega(seqT, seqR, tsT, zg, gsel, c0g, c1g, pos99, gcn_w1, gcn_w2,
          attn_w, ffn_w1, ffn_b1, ffn_w2, ffn_b2, ln_p, fus2_l1w,
          fus2_l1b, fus2_l2w, fus2_l2b, fus_l1w, fus_l1b, fus_l2w,
          fus_l2b, lin2_w, lin2_b):
    out = pl.pallas_call(
        _mega_body,
        grid=(B,),
        in_specs=[
            pl.BlockSpec((1, L, 1), lambda b: (b, 0, 0)),
            pl.BlockSpec((1, 1, L), lambda b: (b, 0, 0)),
            pl.BlockSpec((1, L, 1), lambda b: (b, 0, 0)),
            pl.BlockSpec((1, L, D_MODEL), lambda b: (b, 0, 0)),
            pl.BlockSpec((1, L, D_MODEL), lambda b: (b, 0, 0)),
            pl.BlockSpec((1, 1, D_MODEL), lambda b: (b, 0, 0)),
            pl.BlockSpec((1, 1, D_MODEL), lambda b: (b, 0, 0)),
            _const((L, POS_DIM)),
            _const((D_MODEL, 2 * D_MODEL)),
            _const((2 * D_MODEL, D_MODEL)),
            _const((2, 4, D, D)),
            _const((2, D, D_FF)),
            _const((2, D_FF)),
            _const((2, D_FF, D)),
            _const((2, D)),
            _const((2, 2, 2, D)),
            _const((D_MODEL, D_MODEL)),
            _const((D_MODEL,)),
            _const((D_MODEL, 1)),
            _const((1, 1)),
            _const((D, D)),
            _const((D,)),
            _const((D, 1)),
            _const((1, 1)),
            _const((D, N_NODE)),
            _const((1, N_NODE)),
        ],
        out_specs=pl.BlockSpec((1, L, N_NODE), lambda b: (b, 0, 0)),
        out_shape=jax.ShapeDtypeStruct((B, L, N_NODE), jnp.float32),
    )(seqT, seqR, tsT, zg, gsel, c0g, c1g, pos99, gcn_w1, gcn_w2,
      attn_w, ffn_w1, ffn_b1, ffn_w2, ffn_b2, ln_p, fus2_l1w, fus2_l1b,
      fus2_l2w, fus2_l2b, fus_l1w, fus_l1b, fus_l2w, fus_l2b,
      lin2_w, lin2_b)
    return out.reshape(B * L, N_NODE)


def kernel(input_seq, input_seq_timestamp, tgt_idx, edge_index, emb_table,
           gcn_w1, gcn_w2, global_snaps, cas_snaps, pos_emb, attn_w,
           ffn_w1, ffn_b1, ffn_w2, ffn_b2, ln_p, fus2_l1w, fus2_l1b,
           fus2_l2w, fus2_l2b, fus_l1w, fus_l1b, fus_l2w, fus_l2b,
           lin2_w, lin2_b):
    seq = input_seq[:, :-1]
    ts = input_seq_timestamp[:, :-1]
    src, dst = edge_index[0], edge_index[1]

    zero_big = jnp.zeros((NPAD, D_MODEL), jnp.float32)
    zero_col = jnp.zeros((NPAD,), jnp.float32)
    ones_col = jnp.ones((CH,), jnp.float32)
    deg = _sc_deg(dst, zero_col, ones_col) + 1.0
    norm = jax.lax.rsqrt(deg)
    xn1 = emb_table * norm[:, None]
    agg1 = _sc_agg(xn1, src, dst, zero_big)
    xn2 = (norm * norm)[:, None] * (agg1 + xn1)
    agg2 = _sc_agg(xn2, src, dst, zero_big)
    z = norm[:, None] * (agg2 + xn2)

    zg = z[seq]
    gflat = global_snaps.reshape(3 * N_NODE, D_MODEL)
    gidx = (jnp.clip(ts, 1, 3) - 1) * N_NODE + seq
    gsel = gflat[gidx]
    c0g = cas_snaps[0][tgt_idx][:, None, :]
    c1g = cas_snaps[1][tgt_idx][:, None, :]

    return _mega(seq[:, :, None], seq[:, None, :], ts[:, :, None],
                 zg, gsel, c0g, c1g, pos_emb[:L], gcn_w1, gcn_w2,
                 attn_w, ffn_w1, ffn_b1, ffn_w2, ffn_b2, ln_p,
                 fus2_l1w, fus2_l1b, fus2_l2w, fus2_l2b.reshape(1, 1),
                 fus_l1w, fus_l1b, fus_l2w, fus_l2b.reshape(1, 1),
                 lin2_w, lin2_b.reshape(1, N_NODE))


# speedup vs baseline: 11.1776x; 1.1962x over previous
import functools

import jax
import jax.numpy as jnp
import numpy as np
from jax import lax
from jax.experimental import pallas as pl
from jax.experimental.pallas import tpu as pltpu
from jax.experimental.pallas import tpu_sc as plsc

N_NODE = 10000
D_MODEL = 128
POS_DIM = 8
N_HEADS = 8
B = 16
L = 99
D = D_MODEL + POS_DIM
DK = D // N_HEADS
D_FF = 4 * D


N_EDGES = 160000
NW = 32
EPT = N_EDGES // NW
CH = 312
NCH = EPT // CH
REM = EPT - NCH * CH
NPAD = 10240
RPT = NPAD // 16


def _sc_agg_body(x_hbm, src_hbm, dst_hbm, zero_hbm, out_hbm,
                 idxs, idxd, rows, idxs_r, idxd_r, rows_r, aggs, sem):
    c = lax.axis_index("c")
    s = lax.axis_index("s")
    w = s * 2 + c
    base = w * EPT
    pltpu.sync_copy(zero_hbm.at[pl.ds(s * RPT, RPT)],
                    aggs.at[pl.ds(s * RPT, RPT)])
    plsc.subcore_barrier()

    def body(j, carry):
        off = base + j * CH
        pltpu.sync_copy(src_hbm.at[pl.ds(off, CH)], idxs)
        pltpu.sync_copy(dst_hbm.at[pl.ds(off, CH)], idxd)
        pltpu.async_copy(x_hbm.at[idxs], rows, sem).wait()
        pltpu.sync_copy(rows, aggs.at[idxd], add=True)
        return carry

    lax.fori_loop(0, NCH, body, 0)
    off = base + NCH * CH
    pltpu.sync_copy(src_hbm.at[pl.ds(off, REM)], idxs_r)
    pltpu.sync_copy(dst_hbm.at[pl.ds(off, REM)], idxd_r)
    pltpu.async_copy(x_hbm.at[idxs_r], rows_r, sem).wait()
    pltpu.sync_copy(rows_r, aggs.at[idxd_r], add=True)
    plsc.subcore_barrier()
    pltpu.sync_copy(aggs.at[pl.ds(s * RPT, RPT)],
                    out_hbm.at[c, pl.ds(s * RPT, RPT)])


def _sc_agg(xn, src, dst, zero_big):
    f = pl.kernel(
        _sc_agg_body,
        out_type=jax.ShapeDtypeStruct((2, NPAD, D_MODEL), jnp.float32),
        mesh=plsc.VectorSubcoreMesh(core_axis_name="c", subcore_axis_name="s"),
        scratch_types=[
            pltpu.VMEM((CH,), jnp.int32),
            pltpu.VMEM((CH,), jnp.int32),
            pltpu.VMEM((CH, D_MODEL), jnp.float32),
            pltpu.VMEM((REM,), jnp.int32),
            pltpu.VMEM((REM,), jnp.int32),
            pltpu.VMEM((REM, D_MODEL), jnp.float32),
            pltpu.VMEM_SHARED((NPAD, D_MODEL), jnp.float32),
            pltpu.SemaphoreType.DMA,
        ],
    )
    p = f(xn, src, dst, zero_big)
    return p[0, :N_NODE] + p[1, :N_NODE]


def _sc_deg_body(dst_hbm, zero_hbm, ones_hbm, out_hbm,
                 idxd, ones_v, idxd_r, ones_r, hist):
    c = lax.axis_index("c")
    s = lax.axis_index("s")
    w = s * 2 + c
    base = w * EPT
    pltpu.sync_copy(zero_hbm.at[pl.ds(s * RPT, RPT)],
                    hist.at[pl.ds(s * RPT, RPT)])
    pltpu.sync_copy(ones_hbm, ones_v)
    pltpu.sync_copy(ones_hbm.at[pl.ds(0, REM)], ones_r)
    plsc.subcore_barrier()

    def body(j, carry):
        off = base + j * CH
        pltpu.sync_copy(dst_hbm.at[pl.ds(off, CH)], idxd)
        pltpu.sync_copy(ones_v, hist.at[idxd], add=True)
        return carry

    lax.fori_loop(0, NCH, body, 0)
    off = base + NCH * CH
    pltpu.sync_copy(dst_hbm.at[pl.ds(off, REM)], idxd_r)
    pltpu.sync_copy(ones_r, hist.at[idxd_r], add=True)
    plsc.subcore_barrier()
    pltpu.sync_copy(hist.at[pl.ds(s * RPT, RPT)],
                    out_hbm.at[c, pl.ds(s * RPT, RPT)])


def _sc_deg(dst, zero_col, ones_col):
    f = pl.kernel(
        _sc_deg_body,
        out_type=jax.ShapeDtypeStruct((2, NPAD), jnp.float32),
        mesh=plsc.VectorSubcoreMesh(core_axis_name="c", subcore_axis_name="s"),
        scratch_types=[
            pltpu.VMEM((CH,), jnp.int32),
            pltpu.VMEM((CH,), jnp.float32),
            pltpu.VMEM((REM,), jnp.int32),
            pltpu.VMEM((REM,), jnp.float32),
            pltpu.VMEM_SHARED((NPAD,), jnp.float32),
        ],
    )
    p = f(dst, zero_col, ones_col)
    return p[0, :N_NODE] + p[1, :N_NODE]


def _ln(x, s, b):
    mu = jnp.mean(x, axis=1, keepdims=True)
    var = jnp.mean((x - mu) * (x - mu), axis=1, keepdims=True)
    return (x - mu) * jax.lax.rsqrt(var + 1e-6) * s + b


def _mega_body(seqT, seqR, tsT, zg, gsel, c0r, c1r, pos, gw1, gw2, aw,
               fw1, fb1, fw2, fb2, lnp, f2l1w, f2l1b, f2l2w, f2l2b,
               fl1w, fl1b, fl2w, fl2b, l2w, l2b, out):
    f32 = jnp.float32
    w12 = jnp.dot(gw1[...], gw2[...], preferred_element_type=f32)
    hseq = jnp.dot(zg[0], w12, preferred_element_type=f32)

    st = seqT[0]
    tt = tsT[0]
    nz = st != 0
    is0 = nz & (tt == 0)
    dy = jnp.where(is0, hseq, jnp.where(nz, gsel[0], 0.0))
    casr = jnp.where(tt == 1, c0r[0], c1r[0])
    cas = jnp.where(is0, hseq, jnp.where(nz, casr, 0.0))

    def fuse(a, b, l1w, l1b, l2w_, l2b_):
        sa = jnp.dot(jnp.tanh(jnp.dot(a, l1w, preferred_element_type=f32) + l1b),
                     l2w_, preferred_element_type=f32) + l2b_
        sb = jnp.dot(jnp.tanh(jnp.dot(b, l1w, preferred_element_type=f32) + l1b),
                     l2w_, preferred_element_type=f32) + l2b_
        m = jnp.maximum(sa, sb)
        ea = jnp.exp(sa - m)
        eb = jnp.exp(sb - m)
        den = ea + eb
        return (ea / den) * a + (eb / den) * b

    dyf = fuse(dy, cas, f2l1w[...], f2l1b[...], f2l2w[...], f2l2b[...])
    o99 = pos[...]
    diff = jnp.concatenate([dyf, o99], axis=1)
    fri = jnp.concatenate([hseq, o99], axis=1)

    padR = seqR[0] == 0
    lane = jax.lax.broadcasted_iota(jnp.int32, (1, D), 1)
    scale = 1.0 / np.sqrt(DK)

    def blk(x, bi):
        wq, wk, wv, wo = aw[bi, 0], aw[bi, 1], aw[bi, 2], aw[bi, 3]
        q = jnp.dot(x, wq, preferred_element_type=f32)
        k = jnp.dot(x, wk, preferred_element_type=f32)
        v = jnp.dot(x, wv, preferred_element_type=f32)
        o = jnp.zeros_like(x)
        for h in range(N_HEADS):
            mh = ((lane >= h * DK) & (lane < (h + 1) * DK)).astype(f32)
            s = jax.lax.dot_general(q * mh, k, (((1,), (1,)), ((), ())),
                                    preferred_element_type=f32) * scale
            s = jnp.where(padR, -1e9, s)
            m = jnp.max(s, axis=1, keepdims=True)
            e = jnp.exp(s - m)
            a = e / jnp.sum(e, axis=1, keepdims=True)
            o = o + jnp.dot(a, v * mh, preferred_element_type=f32)
        o = jnp.dot(o, wo, preferred_element_type=f32)
        x1 = _ln(x + o, lnp[bi, 0, 0], lnp[bi, 0, 1])
        ff = jnp.maximum(jnp.dot(x1, fw1[bi], preferred_element_type=f32) + fb1[bi], 0.0)
        ff = jnp.dot(ff, fw2[bi], preferred_element_type=f32) + fb2[bi]
        return _ln(x1 + ff, lnp[bi, 1, 0], lnp[bi, 1, 1])

    datt = blk(diff, 0)
    fatt = blk(fri, 1)
    att = fuse(datt, fatt, fl1w[...], fl1b[...], fl2w[...], fl2b[...])

    acc = jnp.dot(att, l2w[...], preferred_element_type=f32) + l2b[...]
    cols = jax.lax.broadcasted_iota(jnp.int32, (1, N_NODE), 1)
    hit = (st == cols).astype(f32)
    ri = jax.lax.broadcasted_iota(jnp.int32, (L, L), 0)
    ci = jax.lax.broadcasted_iota(jnp.int32, (L, L), 1)
    tril = (ri >= ci).astype(f32)
    cum = jnp.dot(tril, hit, preferred_element_type=f32)
    li = jax.lax.broadcasted_iota(jnp.int32, (L, 1), 0)
    masked = (cum > 0.0) | ((cols == 0) & (li < L - 1))
    out[0] = acc + jnp.where(masked, -1000.0, 0.0)


def _const(shape):
    nd = len(shape)
    return pl.BlockSpec(shape, lambda b, _n=nd: (0,) * _n)


def _mega(seqT, seqR, tsT, zg, gsel, c0g, c1g, pos99, gcn_w1, gcn_w2,
          attn_w, ffn_w1, ffn_b1, ffn_w2, ffn_b2, ln_p, fus2_l1w,
          fus2_l1b, fus2_l2w, fus2_l2b, fus_l1w, fus_l1b, fus_l2w,
          fus_l2b, lin2_w, lin2_b):
    out = pl.pallas_call(
        _mega_body,
        grid=(B,),
        in_specs=[
            pl.BlockSpec((1, L, 1), lambda b: (b, 0, 0)),
            pl.BlockSpec((1, 1, L), lambda b: (b, 0, 0)),
            pl.BlockSpec((1, L, 1), lambda b: (b, 0, 0)),
            pl.BlockSpec((1, L, D_MODEL), lambda b: (b, 0, 0)),
            pl.BlockSpec((1, L, D_MODEL), lambda b: (b, 0, 0)),
            pl.BlockSpec((1, 1, D_MODEL), lambda b: (b, 0, 0)),
            pl.BlockSpec((1, 1, D_MODEL), lambda b: (b, 0, 0)),
            _const((L, POS_DIM)),
            _const((D_MODEL, 2 * D_MODEL)),
            _const((2 * D_MODEL, D_MODEL)),
            _const((2, 4, D, D)),
            _const((2, D, D_FF)),
            _const((2, D_FF)),
            _const((2, D_FF, D)),
            _const((2, D)),
            _const((2, 2, 2, D)),
            _const((D_MODEL, D_MODEL)),
            _const((D_MODEL,)),
            _const((D_MODEL, 1)),
            _const((1, 1)),
            _const((D, D)),
            _const((D,)),
            _const((D, 1)),
            _const((1, 1)),
            _const((D, N_NODE)),
            _const((1, N_NODE)),
        ],
        out_specs=pl.BlockSpec((1, L, N_NODE), lambda b: (b, 0, 0)),
        out_shape=jax.ShapeDtypeStruct((B, L, N_NODE), jnp.float32),
    )(seqT, seqR, tsT, zg, gsel, c0g, c1g, pos99, gcn_w1, gcn_w2,
      attn_w, ffn_w1, ffn_b1, ffn_w2, ffn_b2, ln_p, fus2_l1w, fus2_l1b,
      fus2_l2w, fus2_l2b, fus_l1w, fus_l1b, fus_l2w, fus_l2b,
      lin2_w, lin2_b)
    return out.reshape(B * L, N_NODE)


def kernel(input_seq, input_seq_timestamp, tgt_idx, edge_index, emb_table,
           gcn_w1, gcn_w2, global_snaps, cas_snaps, pos_emb, attn_w,
           ffn_w1, ffn_b1, ffn_w2, ffn_b2, ln_p, fus2_l1w, fus2_l1b,
           fus2_l2w, fus2_l2b, fus_l1w, fus_l1b, fus_l2w, fus_l2b,
           lin2_w, lin2_b):
    seq = input_seq[:, :-1]
    ts = input_seq_timestamp[:, :-1]
    src, dst = edge_index[0], edge_index[1]

    zero_big = jnp.zeros((NPAD, D_MODEL), jnp.float32)
    zero_col = jnp.zeros((NPAD,), jnp.float32)
    ones_col = jnp.ones((CH,), jnp.float32)
    deg = _sc_deg(dst, zero_col, ones_col) + 1.0
    norm = jax.lax.rsqrt(deg)
    xn1 = emb_table * norm[:, None]
    agg1 = _sc_agg(xn1, src, dst, zero_big)
    xn2 = (norm * norm)[:, None] * (agg1 + xn1)
    agg2 = _sc_agg(xn2, src, dst, zero_big)
    z = norm[:, None] * (agg2 + xn2)

    zg = z[seq]
    gflat = global_snaps.reshape(3 * N_NODE, D_MODEL)
    gidx = (jnp.clip(ts, 1, 3) - 1) * N_NODE + seq
    gsel = gflat[gidx]
    c0g = cas_snaps[0][tgt_idx][:, None, :]
    c1g = cas_snaps[1][tgt_idx][:, None, :]

    return _mega(seq[:, :, None], seq[:, None, :], ts[:, :, None],
                 zg, gsel, c0g, c1g, pos_emb[:L], gcn_w1, gcn_w2,
                 attn_w, ffn_w1, ffn_b1, ffn_w2, ffn_b2, ln_p,
                 fus2_l1w, fus2_l1b, fus2_l2w, fus2_l2b.reshape(1, 1),
                 fus_l1w, fus_l1b, fus_l2w, fus_l2b.reshape(1, 1),
                 lin2_w, lin2_b.reshape(1, N_NODE))


# speedup vs baseline: 12.3268x; 1.1028x over previous
import functools

import jax
import jax.numpy as jnp
import numpy as np
from jax import lax
from jax.experimental import pallas as pl
from jax.experimental.pallas import tpu as pltpu
from jax.experimental.pallas import tpu_sc as plsc

N_NODE = 10000
D_MODEL = 128
POS_DIM = 8
N_HEADS = 8
B = 16
L = 99
D = D_MODEL + POS_DIM
DK = D // N_HEADS
D_FF = 4 * D


N_EDGES = 160000
NW = 32
EPT = N_EDGES // NW
CH = 312
NCH = EPT // CH
REM = EPT - NCH * CH
ACH = 160
ANCH = EPT // ACH
AREM = EPT - ANCH * ACH
NPAD = 10240
RPT = NPAD // 16


def _sc_agg_body(x_hbm, src_hbm, dst_hbm, zero_hbm, out_hbm,
                 idxs0, idxd0, rows0, idxs1, idxd1, rows1,
                 idxs_r, idxd_r, rows_r, aggs, sem0, sem1):
    c = lax.axis_index("c")
    s = lax.axis_index("s")
    w = s * 2 + c
    base = w * EPT
    pltpu.sync_copy(zero_hbm.at[pl.ds(s * RPT, RPT)],
                    aggs.at[pl.ds(s * RPT, RPT)])
    plsc.subcore_barrier()

    pltpu.sync_copy(src_hbm.at[pl.ds(base, ACH)], idxs0)
    pltpu.sync_copy(dst_hbm.at[pl.ds(base, ACH)], idxd0)
    pltpu.async_copy(x_hbm.at[idxs0], rows0, sem0)

    def body(i, carry):
        off1 = base + (2 * i + 1) * ACH
        pltpu.sync_copy(src_hbm.at[pl.ds(off1, ACH)], idxs1)
        pltpu.sync_copy(dst_hbm.at[pl.ds(off1, ACH)], idxd1)
        pltpu.async_copy(x_hbm.at[idxs1], rows1, sem1)
        pltpu.make_async_copy(x_hbm.at[idxs0], rows0, sem0).wait()
        pltpu.sync_copy(rows0, aggs.at[idxd0], add=True)
        off0 = base + (2 * i + 2) * ACH
        pltpu.sync_copy(src_hbm.at[pl.ds(off0, ACH)], idxs0)
        pltpu.sync_copy(dst_hbm.at[pl.ds(off0, ACH)], idxd0)
        pltpu.async_copy(x_hbm.at[idxs0], rows0, sem0)
        pltpu.make_async_copy(x_hbm.at[idxs1], rows1, sem1).wait()
        pltpu.sync_copy(rows1, aggs.at[idxd1], add=True)
        return carry

    lax.fori_loop(0, (ANCH - 1) // 2, body, 0)
    pltpu.make_async_copy(x_hbm.at[idxs0], rows0, sem0).wait()
    pltpu.sync_copy(rows0, aggs.at[idxd0], add=True)
    off = base + ANCH * ACH
    pltpu.sync_copy(src_hbm.at[pl.ds(off, AREM)], idxs_r)
    pltpu.sync_copy(dst_hbm.at[pl.ds(off, AREM)], idxd_r)
    pltpu.async_copy(x_hbm.at[idxs_r], rows_r, sem0).wait()
    pltpu.sync_copy(rows_r, aggs.at[idxd_r], add=True)
    plsc.subcore_barrier()
    pltpu.sync_copy(aggs.at[pl.ds(s * RPT, RPT)],
                    out_hbm.at[c, pl.ds(s * RPT, RPT)])


def _sc_agg(xn, src, dst, zero_big):
    f = pl.kernel(
        _sc_agg_body,
        out_type=jax.ShapeDtypeStruct((2, NPAD, D_MODEL), jnp.float32),
        mesh=plsc.VectorSubcoreMesh(core_axis_name="c", subcore_axis_name="s"),
        scratch_types=[
            pltpu.VMEM((ACH,), jnp.int32),
            pltpu.VMEM((ACH,), jnp.int32),
            pltpu.VMEM((ACH, D_MODEL), jnp.float32),
            pltpu.VMEM((ACH,), jnp.int32),
            pltpu.VMEM((ACH,), jnp.int32),
            pltpu.VMEM((ACH, D_MODEL), jnp.float32),
            pltpu.VMEM((AREM,), jnp.int32),
            pltpu.VMEM((AREM,), jnp.int32),
            pltpu.VMEM((AREM, D_MODEL), jnp.float32),
            pltpu.VMEM_SHARED((NPAD, D_MODEL), jnp.float32),
            pltpu.SemaphoreType.DMA,
            pltpu.SemaphoreType.DMA,
        ],
    )
    p = f(xn, src, dst, zero_big)
    return p[0, :N_NODE] + p[1, :N_NODE]


def _sc_deg_body(dst_hbm, zero_hbm, ones_hbm, out_hbm,
                 idxd, ones_v, idxd_r, ones_r, hist):
    c = lax.axis_index("c")
    s = lax.axis_index("s")
    w = s * 2 + c
    base = w * EPT
    pltpu.sync_copy(zero_hbm.at[pl.ds(s * RPT, RPT)],
                    hist.at[pl.ds(s * RPT, RPT)])
    pltpu.sync_copy(ones_hbm, ones_v)
    pltpu.sync_copy(ones_hbm.at[pl.ds(0, REM)], ones_r)
    plsc.subcore_barrier()

    def body(j, carry):
        off = base + j * CH
        pltpu.sync_copy(dst_hbm.at[pl.ds(off, CH)], idxd)
        pltpu.sync_copy(ones_v, hist.at[idxd], add=True)
        return carry

    lax.fori_loop(0, NCH, body, 0)
    off = base + NCH * CH
    pltpu.sync_copy(dst_hbm.at[pl.ds(off, REM)], idxd_r)
    pltpu.sync_copy(ones_r, hist.at[idxd_r], add=True)
    plsc.subcore_barrier()
    pltpu.sync_copy(hist.at[pl.ds(s * RPT, RPT)],
                    out_hbm.at[c, pl.ds(s * RPT, RPT)])


def _sc_deg(dst, zero_col, ones_col):
    f = pl.kernel(
        _sc_deg_body,
        out_type=jax.ShapeDtypeStruct((2, NPAD), jnp.float32),
        mesh=plsc.VectorSubcoreMesh(core_axis_name="c", subcore_axis_name="s"),
        scratch_types=[
            pltpu.VMEM((CH,), jnp.int32),
            pltpu.VMEM((CH,), jnp.float32),
            pltpu.VMEM((REM,), jnp.int32),
            pltpu.VMEM((REM,), jnp.float32),
            pltpu.VMEM_SHARED((NPAD,), jnp.float32),
        ],
    )
    p = f(dst, zero_col, ones_col)
    return p[0, :N_NODE] + p[1, :N_NODE]


def _ln(x, s, b):
    mu = jnp.mean(x, axis=1, keepdims=True)
    var = jnp.mean((x - mu) * (x - mu), axis=1, keepdims=True)
    return (x - mu) * jax.lax.rsqrt(var + 1e-6) * s + b


def _mega_body(seqT, seqR, tsT, zg, gsel, c0r, c1r, pos, gw1, gw2, aw,
               fw1, fb1, fw2, fb2, lnp, f2l1w, f2l1b, f2l2w, f2l2b,
               fl1w, fl1b, fl2w, fl2b, l2w, l2b, out):
    f32 = jnp.float32
    w12 = jnp.dot(gw1[...], gw2[...], preferred_element_type=f32)
    hseq = jnp.dot(zg[0], w12, preferred_element_type=f32)

    st = seqT[0]
    tt = tsT[0]
    nz = st != 0
    is0 = nz & (tt == 0)
    dy = jnp.where(is0, hseq, jnp.where(nz, gsel[0], 0.0))
    casr = jnp.where(tt == 1, c0r[0], c1r[0])
    cas = jnp.where(is0, hseq, jnp.where(nz, casr, 0.0))

    def fuse(a, b, l1w, l1b, l2w_, l2b_):
        sa = jnp.dot(jnp.tanh(jnp.dot(a, l1w, preferred_element_type=f32) + l1b),
                     l2w_, preferred_element_type=f32) + l2b_
        sb = jnp.dot(jnp.tanh(jnp.dot(b, l1w, preferred_element_type=f32) + l1b),
                     l2w_, preferred_element_type=f32) + l2b_
        m = jnp.maximum(sa, sb)
        ea = jnp.exp(sa - m)
        eb = jnp.exp(sb - m)
        den = ea + eb
        return (ea / den) * a + (eb / den) * b

    dyf = fuse(dy, cas, f2l1w[...], f2l1b[...], f2l2w[...], f2l2b[...])
    o99 = pos[...]
    diff = jnp.concatenate([dyf, o99], axis=1)
    fri = jnp.concatenate([hseq, o99], axis=1)

    padR = seqR[0] == 0
    lane = jax.lax.broadcasted_iota(jnp.int32, (1, D), 1)
    scale = 1.0 / np.sqrt(DK)

    def blk(x, bi):
        wq, wk, wv, wo = aw[bi, 0], aw[bi, 1], aw[bi, 2], aw[bi, 3]
        q = jnp.dot(x, wq, preferred_element_type=f32)
        k = jnp.dot(x, wk, preferred_element_type=f32)
        v = jnp.dot(x, wv, preferred_element_type=f32)
        o = jnp.zeros_like(x)
        for h in range(N_HEADS):
            mh = ((lane >= h * DK) & (lane < (h + 1) * DK)).astype(f32)
            s = jax.lax.dot_general(q * mh, k, (((1,), (1,)), ((), ())),
                                    preferred_element_type=f32) * scale
            s = jnp.where(padR, -1e9, s)
            m = jnp.max(s, axis=1, keepdims=True)
            e = jnp.exp(s - m)
            a = e / jnp.sum(e, axis=1, keepdims=True)
            o = o + jnp.dot(a, v * mh, preferred_element_type=f32)
        o = jnp.dot(o, wo, preferred_element_type=f32)
        x1 = _ln(x + o, lnp[bi, 0, 0], lnp[bi, 0, 1])
        ff = jnp.maximum(jnp.dot(x1, fw1[bi], preferred_element_type=f32) + fb1[bi], 0.0)
        ff = jnp.dot(ff, fw2[bi], preferred_element_type=f32) + fb2[bi]
        return _ln(x1 + ff, lnp[bi, 1, 0], lnp[bi, 1, 1])

    datt = blk(diff, 0)
    fatt = blk(fri, 1)
    att = fuse(datt, fatt, fl1w[...], fl1b[...], fl2w[...], fl2b[...])

    acc = jnp.dot(att, l2w[...], preferred_element_type=f32) + l2b[...]
    cols = jax.lax.broadcasted_iota(jnp.int32, (1, N_NODE), 1)
    hit = (st == cols).astype(f32)
    ri = jax.lax.broadcasted_iota(jnp.int32, (L, L), 0)
    ci = jax.lax.broadcasted_iota(jnp.int32, (L, L), 1)
    tril = (ri >= ci).astype(f32)
    cum = jnp.dot(tril, hit, preferred_element_type=f32)
    li = jax.lax.broadcasted_iota(jnp.int32, (L, 1), 0)
    masked = (cum > 0.0) | ((cols == 0) & (li < L - 1))
    out[0] = acc + jnp.where(masked, -1000.0, 0.0)


def _const(shape):
    nd = len(shape)
    return pl.BlockSpec(shape, lambda b, _n=nd: (0,) * _n)


def _mega(seqT, seqR, tsT, zg, gsel, c0g, c1g, pos99, gcn_w1, gcn_w2,
          attn_w, ffn_w1, ffn_b1, ffn_w2, ffn_b2, ln_p, fus2_l1w,
          fus2_l1b, fus2_l2w, fus2_l2b, fus_l1w, fus_l1b, fus_l2w,
          fus_l2b, lin2_w, lin2_b):
    out = pl.pallas_call(
        _mega_body,
        grid=(B,),
        in_specs=[
            pl.BlockSpec((1, L, 1), lambda b: (b, 0, 0)),
            pl.BlockSpec((1, 1, L), lambda b: (b, 0, 0)),
            pl.BlockSpec((1, L, 1), lambda b: (b, 0, 0)),
            pl.BlockSpec((1, L, D_MODEL), lambda b: (b, 0, 0)),
            pl.BlockSpec((1, L, D_MODEL), lambda b: (b, 0, 0)),
            pl.BlockSpec((1, 1, D_MODEL), lambda b: (b, 0, 0)),
            pl.BlockSpec((1, 1, D_MODEL), lambda b: (b, 0, 0)),
            _const((L, POS_DIM)),
            _const((D_MODEL, 2 * D_MODEL)),
            _const((2 * D_MODEL, D_MODEL)),
            _const((2, 4, D, D)),
            _const((2, D, D_FF)),
            _const((2, D_FF)),
            _const((2, D_FF, D)),
            _const((2, D)),
            _const((2, 2, 2, D)),
            _const((D_MODEL, D_MODEL)),
            _const((D_MODEL,)),
            _const((D_MODEL, 1)),
            _const((1, 1)),
            _const((D, D)),
            _const((D,)),
            _const((D, 1)),
            _const((1, 1)),
            _const((D, N_NODE)),
            _const((1, N_NODE)),
        ],
        out_specs=pl.BlockSpec((1, L, N_NODE), lambda b: (b, 0, 0)),
        out_shape=jax.ShapeDtypeStruct((B, L, N_NODE), jnp.float32),
    )(seqT, seqR, tsT, zg, gsel, c0g, c1g, pos99, gcn_w1, gcn_w2,
      attn_w, ffn_w1, ffn_b1, ffn_w2, ffn_b2, ln_p, fus2_l1w, fus2_l1b,
      fus2_l2w, fus2_l2b, fus_l1w, fus_l1b, fus_l2w, fus_l2b,
      lin2_w, lin2_b)
    return out.reshape(B * L, N_NODE)


def kernel(input_seq, input_seq_timestamp, tgt_idx, edge_index, emb_table,
           gcn_w1, gcn_w2, global_snaps, cas_snaps, pos_emb, attn_w,
           ffn_w1, ffn_b1, ffn_w2, ffn_b2, ln_p, fus2_l1w, fus2_l1b,
           fus2_l2w, fus2_l2b, fus_l1w, fus_l1b, fus_l2w, fus_l2b,
           lin2_w, lin2_b):
    seq = input_seq[:, :-1]
    ts = input_seq_timestamp[:, :-1]
    src, dst = edge_index[0], edge_index[1]

    zero_big = jnp.zeros((NPAD, D_MODEL), jnp.float32)
    zero_col = jnp.zeros((NPAD,), jnp.float32)
    ones_col = jnp.ones((CH,), jnp.float32)
    deg = _sc_deg(dst, zero_col, ones_col) + 1.0
    norm = jax.lax.rsqrt(deg)
    xn1 = emb_table * norm[:, None]
    agg1 = _sc_agg(xn1, src, dst, zero_big)
    xn2 = (norm * norm)[:, None] * (agg1 + xn1)
    agg2 = _sc_agg(xn2, src, dst, zero_big)
    z = norm[:, None] * (agg2 + xn2)

    zg = z[seq]
    gflat = global_snaps.reshape(3 * N_NODE, D_MODEL)
    gidx = (jnp.clip(ts, 1, 3) - 1) * N_NODE + seq
    gsel = gflat[gidx]
    c0g = cas_snaps[0][tgt_idx][:, None, :]
    c1g = cas_snaps[1][tgt_idx][:, None, :]

    return _mega(seq[:, :, None], seq[:, None, :], ts[:, :, None],
                 zg, gsel, c0g, c1g, pos_emb[:L], gcn_w1, gcn_w2,
                 attn_w, ffn_w1, ffn_b1, ffn_w2, ffn_b2, ln_p,
                 fus2_l1w, fus2_l1b, fus2_l2w, fus2_l2b.reshape(1, 1),
                 fus_l1w, fus_l1b, fus_l2w, fus_l2b.reshape(1, 1),
                 lin2_w, lin2_b.reshape(1, N_NODE))
